# Initial kernel scaffold; baseline (speedup 1.0000x reference)
import jax
import jax.numpy as jnp
from jax.experimental import pallas as pl


def kernel(node_feats, coordinates, edge_index, params):
    raise NotImplementedError("write your pallas kernel here")


# speedup vs baseline: 2.0641x; 2.0641x over previous
import functools

import jax
import jax.numpy as jnp
from jax import lax
from jax.experimental import pallas as pl
from jax.experimental.pallas import tpu as pltpu
from jax.experimental.pallas import tpu_sc as plsc

C = 128
PADW = 144
NC, NS = 2, 16
NW = NC * NS
KCH = 80

_F32 = jnp.float32


def _sds(shape):
    return jax.ShapeDtypeStruct(shape, _F32)


def _proj_body(nf, cp, wa, wb, ta, tb):
    x = nf[...]
    ta[:, :C] = jnp.dot(x, wa[...], preferred_element_type=_F32)
    tb[:, :C] = jnp.dot(x, wb[...], preferred_element_type=_F32)
    c = cp[...]
    ta[:, C:PADW] = c
    tb[:, C:PADW] = -c


def _proj(nf, cp, wa, wb, n_blk):
    n = nf.shape[0]
    grid = n // n_blk
    return pl.pallas_call(
        _proj_body,
        grid=(grid,),
        in_specs=[
            pl.BlockSpec((n_blk, C), lambda i: (i, 0)),
            pl.BlockSpec((n_blk, 16), lambda i: (i, 0)),
            pl.BlockSpec((C, C), lambda i: (0, 0)),
            pl.BlockSpec((C, C), lambda i: (0, 0)),
        ],
        out_specs=[
            pl.BlockSpec((n_blk, PADW), lambda i: (i, 0)),
            pl.BlockSpec((n_blk, PADW), lambda i: (i, 0)),
        ],
        out_shape=[_sds((n, PADW)), _sds((n, PADW))],
    )(nf, cp, wa, wb)


def _gather_body(ta, tb, sidx, ridx, ga, gb, iv_s, iv_r, bufa, bufb, sem):
    n_edges = sidx.shape[0]
    epw = n_edges // NW
    nchunk = epw // KCH
    wid = lax.axis_index("s") * NC + lax.axis_index("c")

    def chunk(i, carry):
        base = wid * epw + i * KCH
        pltpu.sync_copy(sidx.at[pl.ds(base, KCH)], iv_s)
        pltpu.sync_copy(ridx.at[pl.ds(base, KCH)], iv_r)
        pltpu.async_copy(ta.at[iv_s], bufa, sem).wait()
        pltpu.async_copy(tb.at[iv_r], bufb, sem).wait()
        pltpu.sync_copy(bufa, ga.at[pl.ds(base, KCH)])
        pltpu.sync_copy(bufb, gb.at[pl.ds(base, KCH)])
        return carry

    lax.fori_loop(0, nchunk, chunk, 0)


def _gather(ta, tb, sidx, ridx):
    e = sidx.shape[0]
    mesh = plsc.VectorSubcoreMesh(
        core_axis_name="c", subcore_axis_name="s", num_cores=NC, num_subcores=NS
    )
    return pl.kernel(
        _gather_body,
        out_type=(_sds((e, PADW)), _sds((e, PADW))),
        mesh=mesh,
        compiler_params=pltpu.CompilerParams(use_tc_tiling_on_sc=False),
        scratch_types=[
            pltpu.VMEM((KCH,), jnp.int32),
            pltpu.VMEM((KCH,), jnp.int32),
            pltpu.VMEM((KCH, PADW), _F32),
            pltpu.VMEM((KCH, PADW), _F32),
            pltpu.SemaphoreType.DMA,
        ],
    )(ta, tb, sidx, ridx)


def _edge_body(z0k, cut, ga, gb, wext, b1, w2, b2, wh, bh, wz, bz, g2, att):
    g = ga[...] + gb[...]
    cd = g[:, C:PADW]
    d2 = jnp.sum(cd * cd, axis=1, keepdims=True)
    r = jnp.sqrt(d2)
    cutv = cut[0, 0]
    rbf = jnp.sqrt(2.0 / cutv) * jnp.sin(z0k[...] * (r / cutv)) / r
    ext = jnp.concatenate([r, rbf, jnp.zeros((r.shape[0], 7), _F32)], axis=1)
    h1 = g[:, :C] + b1[...] + jnp.dot(ext, wext[...], preferred_element_type=_F32)
    m = jnp.dot(jax.nn.silu(h1), w2[...], preferred_element_type=_F32) + b2[...]
    hh = jax.nn.silu(jnp.dot(m, wh[...], preferred_element_type=_F32) + bh[...])
    z = jnp.dot(hh, wz[...], preferred_element_type=_F32) + bz[...]
    a = jax.nn.sigmoid(z[:, 0:1])
    g2[:, :C] = m * a
    g2[:, C:PADW] = cd * (z[:, 1:2] / (r + 1.0))
    att[...] = a


def _edge(z0k, cut, ga, gb, wext, b1, w2, b2, wh, bh, wz, bz, e_blk):
    e = ga.shape[0]
    grid = e // e_blk
    full = lambda shape: pl.BlockSpec(shape, lambda i: (0, 0))
    return pl.pallas_call(
        _edge_body,
        grid=(grid,),
        in_specs=[
            full((1, 8)),
            full((1, 1)),
            pl.BlockSpec((e_blk, PADW), lambda i: (i, 0)),
            pl.BlockSpec((e_blk, PADW), lambda i: (i, 0)),
            full((16, C)),
            full((1, C)),
            full((C, C)),
            full((1, C)),
            full((C, 2 * C)),
            full((1, 2 * C)),
            full((2 * C, 2)),
            full((1, 2)),
        ],
        out_specs=[
            pl.BlockSpec((e_blk, PADW), lambda i: (i, 0)),
            pl.BlockSpec((e_blk, 1), lambda i: (i, 0)),
        ],
        out_shape=[_sds((e, PADW)), _sds((e, 1))],
    )(z0k, cut, ga, gb, wext, b1, w2, b2, wh, bh, wz, bz)


def _scatter_body(g2, sidx, zeros, out, iv, buf, acc, sem):
    n_edges = sidx.shape[0]
    epw = n_edges // NW
    nchunk = epw // KCH
    cid = lax.axis_index("c")
    sid = lax.axis_index("s")

    @pl.when(sid == 0)
    def _():
        pltpu.sync_copy(zeros, acc)

    plsc.subcore_barrier()
    wid = cid * NS + sid

    def chunk(i, carry):
        base = wid * epw + i * KCH
        pltpu.sync_copy(sidx.at[pl.ds(base, KCH)], iv)
        pltpu.sync_copy(g2.at[pl.ds(base, KCH)], buf)
        pltpu.sync_copy(buf, acc.at[iv], add=True)
        return carry

    lax.fori_loop(0, nchunk, chunk, 0)
    plsc.subcore_barrier()

    @pl.when(sid == 0)
    def _():
        pltpu.sync_copy(acc, out.at[cid])


def _scatter(g2, sidx, zeros):
    n = zeros.shape[0]
    mesh = plsc.VectorSubcoreMesh(
        core_axis_name="c", subcore_axis_name="s", num_cores=NC, num_subcores=NS
    )
    return pl.kernel(
        _scatter_body,
        out_type=_sds((NC, n, PADW)),
        mesh=mesh,
        compiler_params=pltpu.CompilerParams(use_tc_tiling_on_sc=False),
        scratch_types=[
            pltpu.VMEM((KCH,), jnp.int32),
            pltpu.VMEM((KCH, PADW), _F32),
            pltpu.VMEM_SHARED((n, PADW), _F32),
            pltpu.SemaphoreType.DMA,
        ],
    )(g2, sidx, zeros)


def _node_body(p, nf, cp, wna, wnb, b1, w2, b2, onf, oc):
    m = p[0, :, :C] + p[1, :, :C]
    x = nf[...]
    h = jax.nn.silu(
        jnp.dot(x, wna[...], preferred_element_type=_F32)
        + jnp.dot(m, wnb[...], preferred_element_type=_F32)
        + b1[...]
    )
    onf[...] = jnp.dot(h, w2[...], preferred_element_type=_F32) + b2[...] + x
    oc[...] = cp[...] + p[0, :, C:PADW] + p[1, :, C:PADW]


def _node(part, nf, cp, wna, wnb, b1, w2, b2, n_blk):
    n = nf.shape[0]
    grid = n // n_blk
    full = lambda shape: pl.BlockSpec(shape, lambda i: (0, 0))
    return pl.pallas_call(
        _node_body,
        grid=(grid,),
        in_specs=[
            pl.BlockSpec((NC, n_blk, PADW), lambda i: (0, i, 0)),
            pl.BlockSpec((n_blk, C), lambda i: (i, 0)),
            pl.BlockSpec((n_blk, 16), lambda i: (i, 0)),
            full((C, C)),
            full((C, C)),
            full((1, C)),
            full((C, C)),
            full((1, C)),
        ],
        out_specs=[
            pl.BlockSpec((n_blk, C), lambda i: (i, 0)),
            pl.BlockSpec((n_blk, 16), lambda i: (i, 0)),
        ],
        out_shape=[_sds((n, C)), _sds((n, 16))],
    )(part, nf, cp, wna, wnb, b1, w2, b2)


def kernel(node_feats, coordinates, edge_index, params):
    n = node_feats.shape[0]
    pe, pn, pa, px = params["phi_e"], params["phi_n"], params["att"], params["phi_x"]
    cut = params["bessel_cut_off"].reshape(1, 1).astype(_F32)
    z0k = params["z_0k"].reshape(1, -1).astype(_F32)
    nrbf = z0k.shape[1]

    w1 = pe["W1"]
    w1a, w1b = w1[:C], w1[C : 2 * C]
    wext = jnp.zeros((16, C), _F32).at[: nrbf + 1].set(w1[2 * C :])
    b1 = pe["b1"].reshape(1, C)
    w2, b2 = pe["W2"], pe["b2"].reshape(1, C)
    wh = jnp.concatenate([pa["W1"], px["W1"]], axis=1)
    bh = jnp.concatenate([pa["b1"], px["b1"]]).reshape(1, 2 * C)
    wz = (
        jnp.zeros((2 * C, 2), _F32)
        .at[:C, 0:1].set(pa["W2"])
        .at[C:, 1:2].set(px["W2"])
    )
    bz = jnp.concatenate([pa["b2"], px["b2"]]).reshape(1, 2)

    cp = jnp.pad(coordinates, ((0, 0), (0, 16 - coordinates.shape[1])))
    sidx, ridx = edge_index[0], edge_index[1]

    ta, tb = _proj(node_feats, cp, w1a, w1b, n_blk=2000)
    ga, gb = _gather(ta, tb, sidx, ridx)
    g2, att = _edge(z0k, cut, ga, gb, wext, b1, w2, b2, wh, bh, wz, bz, e_blk=2560)
    zeros = jnp.zeros((n, PADW), _F32)
    part = _scatter(g2, sidx, zeros)
    onf, oc = _node(
        part, node_feats, cp, pn["W1"][:C], pn["W1"][C:],
        pn["b1"].reshape(1, C), pn["W2"], pn["b2"].reshape(1, C), n_blk=2000,
    )
    return onf, oc[:, :3], att


# speedup vs baseline: 3.0648x; 1.4848x over previous
import jax
import jax.numpy as jnp
from jax import lax
from jax.experimental import pallas as pl
from jax.experimental.pallas import tpu as pltpu
from jax.experimental.pallas import tpu_sc as plsc

C = 128
CW = 16
PK = 8
NC, NS = 2, 16
NW = NC * NS
KCH = 80

_F32 = jnp.float32


def _sds(shape):
    return jax.ShapeDtypeStruct(shape, _F32)


def _sc_mesh():
    return plsc.VectorSubcoreMesh(
        core_axis_name="c", subcore_axis_name="s", num_cores=NC, num_subcores=NS
    )


_SC_PARAMS = pltpu.CompilerParams(use_tc_tiling_on_sc=False)


def _proj_body(nf, wa, wb, pa, pb):
    x = nf[...]
    pa[...] = jnp.dot(x, wa[...], preferred_element_type=_F32)
    pb[...] = jnp.dot(x, wb[...], preferred_element_type=_F32)


def _proj(nf, wa, wb, n_blk):
    n = nf.shape[0]
    return pl.pallas_call(
        _proj_body,
        grid=(n // n_blk,),
        in_specs=[
            pl.BlockSpec((n_blk, C), lambda i: (i, 0)),
            pl.BlockSpec((C, C), lambda i: (0, 0)),
            pl.BlockSpec((C, C), lambda i: (0, 0)),
        ],
        out_specs=[
            pl.BlockSpec((n_blk, C), lambda i: (i, 0)),
            pl.BlockSpec((n_blk, C), lambda i: (i, 0)),
        ],
        out_shape=[_sds((n, C)), _sds((n, C))],
    )(nf, wa, wb)


def _gather_body(pa, pb, cp, sidx, ridx, g, gc,
                 iv_s, iv_r, bufa, bufb, bufca, bufcb, bufc, sem, semc):
    n_edges = sidx.shape[0]
    epw = n_edges // NW
    nchunk = epw // KCH
    wid = lax.axis_index("s") * NC + lax.axis_index("c")

    def chunk(i, carry):
        base = wid * epw + i * KCH
        pltpu.sync_copy(sidx.at[pl.ds(base, KCH)], iv_s)
        pltpu.sync_copy(ridx.at[pl.ds(base, KCH)], iv_r)
        da = pltpu.async_copy(pa.at[iv_s], bufa, sem)
        db = pltpu.async_copy(pb.at[iv_r], bufb, sem)
        dca = pltpu.async_copy(cp.at[iv_s], bufca, semc)
        dcb = pltpu.async_copy(cp.at[iv_r], bufcb, semc)
        da.wait()
        db.wait()
        dca.wait()
        dcb.wait()

        def addrow(t, c2):
            for j in range(C // 16):
                sl = pl.ds(j * 16, 16)
                bufa[t, sl] = bufa[t, sl] + bufb[t, sl]
            bufc[t, pl.ds(0, CW)] = bufca[t, :] - bufcb[t, :]
            return c2

        lax.fori_loop(0, KCH, addrow, 0)
        pltpu.sync_copy(bufa, g.at[pl.ds(base, KCH)])
        pltpu.sync_copy(bufc, gc.at[pl.ds(base, KCH)])
        return carry

    lax.fori_loop(0, nchunk, chunk, 0)


def _gather(pa, pb, cp, sidx, ridx):
    e = sidx.shape[0]
    return pl.kernel(
        _gather_body,
        out_type=(_sds((e, C)), _sds((e, C))),
        mesh=_sc_mesh(),
        compiler_params=_SC_PARAMS,
        scratch_types=[
            pltpu.VMEM((KCH,), jnp.int32),
            pltpu.VMEM((KCH,), jnp.int32),
            pltpu.VMEM((KCH, C), _F32),
            pltpu.VMEM((KCH, C), _F32),
            pltpu.VMEM((KCH, CW), _F32),
            pltpu.VMEM((KCH, CW), _F32),
            pltpu.VMEM((KCH, C), _F32),
            pltpu.SemaphoreType.DMA,
            pltpu.SemaphoreType.DMA,
        ],
    )(pa, pb, cp, sidx, ridx)


def _edge_body(z0k, cut, g, gc, wext, b1, w2, b2, wh, bh, wz, bz, g2, gd, att):
    e_blk = g.shape[0]
    cd = gc[:, :CW]
    d2 = jnp.sum(cd * cd, axis=1, keepdims=True)
    r = jnp.sqrt(d2)
    cutv = cut[0, 0]
    rbf = jnp.sqrt(2.0 / cutv) * jnp.sin(z0k[...] * (r / cutv)) / r
    ext = jnp.concatenate([r, rbf, jnp.zeros((e_blk, 7), _F32)], axis=1)
    h1 = g[...] + b1[...] + jnp.dot(ext, wext[...], preferred_element_type=_F32)
    m = jnp.dot(jax.nn.silu(h1), w2[...], preferred_element_type=_F32) + b2[...]
    hh = jax.nn.silu(jnp.dot(m, wh[...], preferred_element_type=_F32) + bh[...])
    z = jnp.dot(hh, wz[...], preferred_element_type=_F32) + bz[...]
    a = jax.nn.sigmoid(z[:, 0:1])
    g2[...] = m * a
    gd[...] = cd * (z[:, 1:2] / (r + 1.0))
    att[...] = a


def _edge(z0k, cut, g, gc, wext, b1, w2, b2, wh, bh, wz, bz, e_blk):
    e = g.shape[0]
    full = lambda shape: pl.BlockSpec(shape, lambda i: (0, 0))
    return pl.pallas_call(
        _edge_body,
        grid=(e // e_blk,),
        in_specs=[
            full((1, 8)),
            full((1, 1)),
            pl.BlockSpec((e_blk, C), lambda i: (i, 0)),
            pl.BlockSpec((e_blk, C), lambda i: (i, 0)),
            full((16, C)),
            full((1, C)),
            full((C, C)),
            full((1, C)),
            full((C, 2 * C)),
            full((1, 2 * C)),
            full((2 * C, 2)),
            full((1, 2)),
        ],
        out_specs=[
            pl.BlockSpec((e_blk, C), lambda i: (i, 0)),
            pl.BlockSpec((e_blk, CW), lambda i: (i, 0)),
            pl.BlockSpec((e_blk, 1), lambda i: (i, 0)),
        ],
        out_shape=[_sds((e, C)), _sds((e, CW)), _sds((e, 1))],
    )(z0k, cut, g, gc, wext, b1, w2, b2, wh, bh, wz, bz)


def _scatter_body(g2, gd, sidx, zf, zc, outf, outc,
                  iv, buff, bufd, accf, accc, sem):
    n_edges = sidx.shape[0]
    cid = lax.axis_index("c")
    sid = lax.axis_index("s")

    @pl.when(sid == 0)
    def _():
        pltpu.sync_copy(zf, accf)
        pltpu.sync_copy(zc, accc)

    plsc.subcore_barrier()
    wid = cid * NS + sid
    epw = n_edges // NW
    nchunk = epw // KCH

    def chunk(i, carry):
        base = wid * epw + i * KCH
        pltpu.sync_copy(sidx.at[pl.ds(base, KCH)], iv)
        df = pltpu.async_copy(g2.at[pl.ds(base, KCH)], buff, sem)
        dd = pltpu.async_copy(gd.at[pl.ds(base, KCH)], bufd, sem)
        df.wait()
        dd.wait()
        pltpu.sync_copy(buff, accf.at[iv], add=True)
        pltpu.sync_copy(bufd, accc.at[iv], add=True)
        return carry

    lax.fori_loop(0, nchunk, chunk, 0)
    plsc.subcore_barrier()

    @pl.when(sid == 0)
    def _():
        pltpu.sync_copy(accf, outf.at[cid])
        pltpu.sync_copy(accc, outc.at[cid])


def _scatter(g2, gd, sidx, zf, zc):
    n = zf.shape[0]
    return pl.kernel(
        _scatter_body,
        out_type=(_sds((NC, n, C)), _sds((NC, n, CW))),
        mesh=_sc_mesh(),
        compiler_params=_SC_PARAMS,
        scratch_types=[
            pltpu.VMEM((KCH,), jnp.int32),
            pltpu.VMEM((KCH, C), _F32),
            pltpu.VMEM((KCH, CW), _F32),
            pltpu.VMEM_SHARED((n, C), _F32),
            pltpu.VMEM_SHARED((n, CW), _F32),
            pltpu.SemaphoreType.DMA,
        ],
    )(g2, gd, sidx, zf, zc)


def _node_body(pf, pc, nf, co, wna, wnb, b1, w2, b2, onf, oc):
    m = pf[0] + pf[1]
    x = nf[...]
    h = jax.nn.silu(
        jnp.dot(x, wna[...], preferred_element_type=_F32)
        + jnp.dot(m, wnb[...], preferred_element_type=_F32)
        + b1[...]
    )
    onf[...] = jnp.dot(h, w2[...], preferred_element_type=_F32) + b2[...] + x
    dc = pc[0] + pc[1]
    oc[...] = co[...] + dc[:, :3]


def _node(pf, pc, nf, co, wna, wnb, b1, w2, b2, n_blk):
    n = nf.shape[0]
    full = lambda shape: pl.BlockSpec(shape, lambda i: (0, 0))
    return pl.pallas_call(
        _node_body,
        grid=(n // n_blk,),
        in_specs=[
            pl.BlockSpec((NC, n_blk, C), lambda i: (0, i, 0)),
            pl.BlockSpec((NC, n_blk, CW), lambda i: (0, i, 0)),
            pl.BlockSpec((n_blk, C), lambda i: (i, 0)),
            pl.BlockSpec((n_blk, 3), lambda i: (i, 0)),
            full((C, C)),
            full((C, C)),
            full((1, C)),
            full((C, C)),
            full((1, C)),
        ],
        out_specs=[
            pl.BlockSpec((n_blk, C), lambda i: (i, 0)),
            pl.BlockSpec((n_blk, 3), lambda i: (i, 0)),
        ],
        out_shape=[_sds((n, C)), _sds((n, 3))],
    )(pf, pc, nf, co, wna, wnb, b1, w2, b2)


def kernel(node_feats, coordinates, edge_index, params):
    n = node_feats.shape[0]
    pe, pn, pa, px = params["phi_e"], params["phi_n"], params["att"], params["phi_x"]
    cut = params["bessel_cut_off"].reshape(1, 1).astype(_F32)
    z0k = params["z_0k"].reshape(1, -1).astype(_F32)
    nrbf = z0k.shape[1]

    w1 = pe["W1"]
    w1a, w1b = w1[:C], w1[C : 2 * C]
    wext = jnp.zeros((16, C), _F32).at[: nrbf + 1].set(w1[2 * C :])
    b1 = pe["b1"].reshape(1, C)
    w2, b2 = pe["W2"], pe["b2"].reshape(1, C)
    wh = jnp.concatenate([pa["W1"], px["W1"]], axis=1)
    bh = jnp.concatenate([pa["b1"], px["b1"]]).reshape(1, 2 * C)
    wz = (
        jnp.zeros((2 * C, 2), _F32)
        .at[:C, 0:1].set(pa["W2"])
        .at[C:, 1:2].set(px["W2"])
    )
    bz = jnp.concatenate([pa["b2"], px["b2"]]).reshape(1, 2)

    cp = jnp.pad(coordinates, ((0, 0), (0, CW - coordinates.shape[1])))
    sidx, ridx = edge_index[0], edge_index[1]

    pa_t, pb_t = _proj(node_feats, w1a, w1b, n_blk=2000)
    g, gc = _gather(pa_t, pb_t, cp, sidx, ridx)
    g2, gd, att = _edge(
        z0k, cut, g, gc, wext, b1, w2, b2, wh, bh, wz, bz, e_blk=2560
    )
    zf = jnp.zeros((n, C), _F32)
    zc = jnp.zeros((n, CW), _F32)
    pf, pc = _scatter(g2, gd, sidx, zf, zc)
    onf, oc = _node(
        pf, pc, node_feats, coordinates, pn["W1"][:C], pn["W1"][C:],
        pn["b1"].reshape(1, C), pn["W2"], pn["b2"].reshape(1, C), n_blk=2000,
    )
    return onf, oc, att


# speedup vs baseline: 4.2956x; 1.4016x over previous
import jax
import jax.numpy as jnp
from jax import lax
from jax.experimental import pallas as pl
from jax.experimental.pallas import tpu as pltpu
from jax.experimental.pallas import tpu_sc as plsc

C = 128
CW = 16
PK = 8
NC, NS = 2, 16
NW = NC * NS
KCH = 80
_NSLICE = 5

_F32 = jnp.float32


def _sds(shape):
    return jax.ShapeDtypeStruct(shape, _F32)


def _sc_mesh():
    return plsc.VectorSubcoreMesh(
        core_axis_name="c", subcore_axis_name="s", num_cores=NC, num_subcores=NS
    )


_SC_PARAMS = pltpu.CompilerParams(use_tc_tiling_on_sc=False)


def _proj_body(nf, wa, wb, pa, pb):
    x = nf[...]
    pa[...] = jnp.dot(x, wa[...], preferred_element_type=_F32)
    pb[...] = jnp.dot(x, wb[...], preferred_element_type=_F32)


def _proj(nf, wa, wb, n_blk):
    n = nf.shape[0]
    return pl.pallas_call(
        _proj_body,
        grid=(n // n_blk,),
        in_specs=[
            pl.BlockSpec((n_blk, C), lambda i: (i, 0)),
            pl.BlockSpec((C, C), lambda i: (0, 0)),
            pl.BlockSpec((C, C), lambda i: (0, 0)),
        ],
        out_specs=[
            pl.BlockSpec((n_blk, C), lambda i: (i, 0)),
            pl.BlockSpec((n_blk, C), lambda i: (i, 0)),
        ],
        out_shape=[_sds((n, C)), _sds((n, C))],
    )(nf, wa, wb)


def _gather_body(pa, pb, cp, sidx, ridx, g, gc,
                 iv_s, iv_r, bufa, bufb, bufca, bufcb, bufc, sem, semc):
    n_edges = sidx.shape[0]
    epw = n_edges // NW
    nchunk = epw // KCH
    wid = lax.axis_index("s") * NC + lax.axis_index("c")

    def chunk(i, carry):
        base = wid * epw + i * KCH
        pltpu.sync_copy(sidx.at[pl.ds(base, KCH)], iv_s)
        pltpu.sync_copy(ridx.at[pl.ds(base, KCH)], iv_r)
        da = pltpu.async_copy(pa.at[iv_s], bufa, sem)
        db = pltpu.async_copy(pb.at[iv_r], bufb, sem)
        dca = pltpu.async_copy(cp.at[iv_s], bufca, semc)
        dcb = pltpu.async_copy(cp.at[iv_r], bufcb, semc)
        da.wait()
        db.wait()
        dca.wait()
        dcb.wait()

        def addrow(t, c2):
            for j in range(C // 16):
                sl = pl.ds(j * 16, 16)
                bufa[t, sl] = bufa[t, sl] + bufb[t, sl]
            bufc[t, pl.ds(0, CW)] = bufca[t, :] - bufcb[t, :]
            return c2

        lax.fori_loop(0, KCH, addrow, 0)
        pltpu.sync_copy(bufa, g.at[pl.ds(base, KCH)])
        pltpu.sync_copy(bufc, gc.at[pl.ds(base, KCH)])
        return carry

    lax.fori_loop(0, nchunk, chunk, 0)


def _gather(pa, pb, cp, sidx, ridx):
    e = sidx.shape[0]
    return pl.kernel(
        _gather_body,
        out_type=(_sds((e, C)), _sds((e, C))),
        mesh=_sc_mesh(),
        compiler_params=_SC_PARAMS,
        scratch_types=[
            pltpu.VMEM((KCH,), jnp.int32),
            pltpu.VMEM((KCH,), jnp.int32),
            pltpu.VMEM((KCH, C), _F32),
            pltpu.VMEM((KCH, C), _F32),
            pltpu.VMEM((KCH, CW), _F32),
            pltpu.VMEM((KCH, CW), _F32),
            pltpu.VMEM((KCH, C), _F32),
            pltpu.SemaphoreType.DMA,
            pltpu.SemaphoreType.DMA,
        ],
    )(pa, pb, cp, sidx, ridx)


def _edge_body(z0k, cut, g, gc, wext, b1, w2, b2, wh, bh, wz, bz, g2, gd, att):
    e_blk = g.shape[0]
    cd = gc[:, :CW]
    d2 = jnp.sum(cd * cd, axis=1, keepdims=True)
    r = jnp.sqrt(d2)
    cutv = cut[0, 0]
    rbf = jnp.sqrt(2.0 / cutv) * jnp.sin(z0k[...] * (r / cutv)) / r
    ext = jnp.concatenate([r, rbf, jnp.zeros((e_blk, 7), _F32)], axis=1)
    h1 = g[...] + b1[...] + jnp.dot(ext, wext[...], preferred_element_type=_F32)
    m = jnp.dot(jax.nn.silu(h1), w2[...], preferred_element_type=_F32) + b2[...]
    hh = jax.nn.silu(jnp.dot(m, wh[...], preferred_element_type=_F32) + bh[...])
    z = jnp.dot(hh, wz[...], preferred_element_type=_F32) + bz[...]
    a = jax.nn.sigmoid(z[:, 0:1])
    g2[...] = m * a
    gd[...] = cd * (z[:, 1:2] / (r + 1.0))
    att[...] = a


def _edge(z0k, cut, g, gc, wext, b1, w2, b2, wh, bh, wz, bz, e_blk):
    e = g.shape[0]
    full = lambda shape: pl.BlockSpec(shape, lambda i: (0, 0))
    return pl.pallas_call(
        _edge_body,
        grid=(e // e_blk,),
        in_specs=[
            full((1, 8)),
            full((1, 1)),
            pl.BlockSpec((e_blk, C), lambda i: (i, 0)),
            pl.BlockSpec((e_blk, C), lambda i: (i, 0)),
            full((16, C)),
            full((1, C)),
            full((C, C)),
            full((1, C)),
            full((C, 2 * C)),
            full((1, 2 * C)),
            full((2 * C, 2)),
            full((1, 2)),
        ],
        out_specs=[
            pl.BlockSpec((e_blk, C), lambda i: (i, 0)),
            pl.BlockSpec((e_blk, CW), lambda i: (i, 0)),
            pl.BlockSpec((e_blk, 1), lambda i: (i, 0)),
        ],
        out_shape=[_sds((e, C)), _sds((e, CW)), _sds((e, 1))],
    )(z0k, cut, g, gc, wext, b1, w2, b2, wh, bh, wz, bz)


def _scatter_body(g2, gd, sidx, zf, zc, outf, outc,
                  iv, buff, bufd, accf, accc, sem):
    n_edges = sidx.shape[0]
    cid = lax.axis_index("c")
    sid = lax.axis_index("s")

    @pl.when(sid == 0)
    def _():
        pltpu.sync_copy(zf, accf)
        pltpu.sync_copy(zc, accc)

    plsc.subcore_barrier()
    wid = cid * NS + sid
    epw = n_edges // NW
    nchunk = epw // KCH

    def chunk(i, carry):
        base = wid * epw + i * KCH
        pltpu.sync_copy(sidx.at[pl.ds(base, KCH)], iv)
        df = pltpu.async_copy(g2.at[pl.ds(base, KCH)], buff, sem)
        dd = pltpu.async_copy(gd.at[pl.ds(base, KCH)], bufd, sem)
        df.wait()
        dd.wait()
        pltpu.sync_copy(buff, accf.at[iv], add=True)
        pltpu.sync_copy(bufd, accc.at[iv], add=True)
        return carry

    lax.fori_loop(0, nchunk, chunk, 0)
    plsc.subcore_barrier()

    @pl.when(sid == 0)
    def _():
        pltpu.sync_copy(accf, outf.at[cid])
        pltpu.sync_copy(accc, outc.at[cid])


def _scatter(g2, gd, sidx, zf, zc):
    n = zf.shape[0]
    return pl.kernel(
        _scatter_body,
        out_type=(_sds((NC, n, C)), _sds((NC, n, CW))),
        mesh=_sc_mesh(),
        compiler_params=_SC_PARAMS,
        scratch_types=[
            pltpu.VMEM((KCH,), jnp.int32),
            pltpu.VMEM((KCH, C), _F32),
            pltpu.VMEM((KCH, CW), _F32),
            pltpu.VMEM_SHARED((n, C), _F32),
            pltpu.VMEM_SHARED((n, CW), _F32),
            pltpu.SemaphoreType.DMA,
        ],
    )(g2, gd, sidx, zf, zc)


def _node_body(*refs):
    pfs = refs[:_NSLICE]
    pcs = refs[_NSLICE : 2 * _NSLICE]
    nf, co, wna, wnb, b1, w2, b2, onf, oc = refs[2 * _NSLICE :]
    m = pfs[0][0] + pfs[0][1]
    for p in pfs[1:]:
        m = m + p[0] + p[1]
    dc = pcs[0][0] + pcs[0][1]
    for p in pcs[1:]:
        dc = dc + p[0] + p[1]
    x = nf[...]
    h = jax.nn.silu(
        jnp.dot(x, wna[...], preferred_element_type=_F32)
        + jnp.dot(m, wnb[...], preferred_element_type=_F32)
        + b1[...]
    )
    onf[...] = jnp.dot(h, w2[...], preferred_element_type=_F32) + b2[...] + x
    oc[...] = co[...] + dc[:, :3]


def _node(pfs, pcs, nf, co, wna, wnb, b1, w2, b2, n_blk):
    n = nf.shape[0]
    full = lambda shape: pl.BlockSpec(shape, lambda i: (0, 0))
    return pl.pallas_call(
        _node_body,
        grid=(n // n_blk,),
        in_specs=(
            [pl.BlockSpec((NC, n_blk, C), lambda i: (0, i, 0))] * _NSLICE
            + [pl.BlockSpec((NC, n_blk, CW), lambda i: (0, i, 0))] * _NSLICE
            + [
                pl.BlockSpec((n_blk, C), lambda i: (i, 0)),
                pl.BlockSpec((n_blk, 3), lambda i: (i, 0)),
                full((C, C)),
                full((C, C)),
                full((1, C)),
                full((C, C)),
                full((1, C)),
            ]
        ),
        out_specs=[
            pl.BlockSpec((n_blk, C), lambda i: (i, 0)),
            pl.BlockSpec((n_blk, 3), lambda i: (i, 0)),
        ],
        out_shape=[_sds((n, C)), _sds((n, 3))],
    )(*pfs, *pcs, nf, co, wna, wnb, b1, w2, b2)


def kernel(node_feats, coordinates, edge_index, params):
    n = node_feats.shape[0]
    pe, pn, pa, px = params["phi_e"], params["phi_n"], params["att"], params["phi_x"]
    cut = params["bessel_cut_off"].reshape(1, 1).astype(_F32)
    z0k = params["z_0k"].reshape(1, -1).astype(_F32)
    nrbf = z0k.shape[1]

    w1 = pe["W1"]
    w1a, w1b = w1[:C], w1[C : 2 * C]
    wext = jnp.zeros((16, C), _F32).at[: nrbf + 1].set(w1[2 * C :])
    b1 = pe["b1"].reshape(1, C)
    w2, b2 = pe["W2"], pe["b2"].reshape(1, C)
    wh = jnp.concatenate([pa["W1"], px["W1"]], axis=1)
    bh = jnp.concatenate([pa["b1"], px["b1"]]).reshape(1, 2 * C)
    wz = (
        jnp.zeros((2 * C, 2), _F32)
        .at[:C, 0:1].set(pa["W2"])
        .at[C:, 1:2].set(px["W2"])
    )
    bz = jnp.concatenate([pa["b2"], px["b2"]]).reshape(1, 2)

    cp = jnp.pad(coordinates, ((0, 0), (0, CW - coordinates.shape[1])))
    sidx, ridx = edge_index[0], edge_index[1]
    e = sidx.shape[0]
    es = e // _NSLICE

    pa_t, pb_t = _proj(node_feats, w1a, w1b, n_blk=2000)
    zf = jnp.zeros((n, C), _F32)
    zc = jnp.zeros((n, CW), _F32)
    pfs, pcs, atts = [], [], []
    for k in range(_NSLICE):
        sl = slice(k * es, (k + 1) * es)
        g, gck = _gather(pa_t, pb_t, cp, sidx[sl], ridx[sl])
        g2, gd, att = _edge(
            z0k, cut, g, gck, wext, b1, w2, b2, wh, bh, wz, bz, e_blk=2560
        )
        pf, pc = _scatter(g2, gd, sidx[sl], zf, zc)
        pfs.append(pf)
        pcs.append(pc)
        atts.append(att)
    onf, oc = _node(
        pfs, pcs, node_feats, coordinates, pn["W1"][:C], pn["W1"][C:],
        pn["b1"].reshape(1, C), pn["W2"], pn["b2"].reshape(1, C), n_blk=1000,
    )
    return onf, oc, jnp.concatenate(atts, axis=0)


# speedup vs baseline: 4.3495x; 1.0125x over previous
import jax
import jax.numpy as jnp
from jax import lax
from jax.experimental import pallas as pl
from jax.experimental.pallas import tpu as pltpu
from jax.experimental.pallas import tpu_sc as plsc

C = 128
CW = 16
PK = 8
NC, NS = 2, 16
NW = NC * NS
KCH = 40
_NSLICE = 5

_F32 = jnp.float32


def _sds(shape):
    return jax.ShapeDtypeStruct(shape, _F32)


def _sc_mesh():
    return plsc.VectorSubcoreMesh(
        core_axis_name="c", subcore_axis_name="s", num_cores=NC, num_subcores=NS
    )


_SC_PARAMS = pltpu.CompilerParams(use_tc_tiling_on_sc=False)


def _proj_body(nf, wa, wb, pa, pb):
    x = nf[...]
    pa[...] = jnp.dot(x, wa[...], preferred_element_type=_F32)
    pb[...] = jnp.dot(x, wb[...], preferred_element_type=_F32)


def _proj(nf, wa, wb, n_blk):
    n = nf.shape[0]
    return pl.pallas_call(
        _proj_body,
        grid=(n // n_blk,),
        in_specs=[
            pl.BlockSpec((n_blk, C), lambda i: (i, 0)),
            pl.BlockSpec((C, C), lambda i: (0, 0)),
            pl.BlockSpec((C, C), lambda i: (0, 0)),
        ],
        out_specs=[
            pl.BlockSpec((n_blk, C), lambda i: (i, 0)),
            pl.BlockSpec((n_blk, C), lambda i: (i, 0)),
        ],
        out_shape=[_sds((n, C)), _sds((n, C))],
    )(nf, wa, wb)


def _gather_body(pa, pb, cp, sidx, ridx, g, gc, *scr):
    n_edges = sidx.shape[0]
    epw = n_edges // NW
    nchunk = epw // KCH
    wid = lax.axis_index("s") * NC + lax.axis_index("c")
    sets = (scr[:8], scr[8:])

    def idx_fire(c, s):
        iv_s, iv_r, bufa, bufb, bufca, bufcb, _, sem = s
        base = wid * epw + c * KCH
        pltpu.sync_copy(sidx.at[pl.ds(base, KCH)], iv_s)
        pltpu.sync_copy(ridx.at[pl.ds(base, KCH)], iv_r)
        pltpu.async_copy(pa.at[iv_s], bufa, sem)
        pltpu.async_copy(pb.at[iv_r], bufb, sem)
        pltpu.async_copy(cp.at[iv_s], bufca, sem)
        pltpu.async_copy(cp.at[iv_r], bufcb, sem)

    def process(c, s):
        iv_s, iv_r, bufa, bufb, bufca, bufcb, bufc, sem = s
        base = wid * epw + c * KCH
        pltpu.make_async_copy(pa.at[iv_s], bufa, sem).wait()
        pltpu.make_async_copy(pb.at[iv_r], bufb, sem).wait()
        pltpu.make_async_copy(cp.at[iv_s], bufca, sem).wait()
        pltpu.make_async_copy(cp.at[iv_r], bufcb, sem).wait()

        def addrow(t, c2):
            for j in range(C // 16):
                sl = pl.ds(j * 16, 16)
                bufa[t, sl] = bufa[t, sl] + bufb[t, sl]
            bufc[t, pl.ds(0, CW)] = bufca[t, :] - bufcb[t, :]
            return c2

        lax.fori_loop(0, KCH, addrow, 0)
        pltpu.sync_copy(bufa, g.at[pl.ds(base, KCH)])
        pltpu.sync_copy(bufc, gc.at[pl.ds(base, KCH)])

    idx_fire(0, sets[0])

    def chunk(i, carry):
        for par in range(2):
            @pl.when((i & 1) == par)
            def _():
                @pl.when(i + 1 < nchunk)
                def _():
                    idx_fire(i + 1, sets[1 - par])

                process(i, sets[par])

        return carry

    lax.fori_loop(0, nchunk, chunk, 0)


def _gather(pa, pb, cp, sidx, ridx):
    e = sidx.shape[0]
    one_set = [
        pltpu.VMEM((KCH,), jnp.int32),
        pltpu.VMEM((KCH,), jnp.int32),
        pltpu.VMEM((KCH, C), _F32),
        pltpu.VMEM((KCH, C), _F32),
        pltpu.VMEM((KCH, CW), _F32),
        pltpu.VMEM((KCH, CW), _F32),
        pltpu.VMEM((KCH, C), _F32),
        pltpu.SemaphoreType.DMA,
    ]
    return pl.kernel(
        _gather_body,
        out_type=(_sds((e, C)), _sds((e, C))),
        mesh=_sc_mesh(),
        compiler_params=_SC_PARAMS,
        scratch_types=one_set + one_set,
    )(pa, pb, cp, sidx, ridx)


def _edge_body(z0k, cut, g, gc, wext, b1, w2, b2, wh, bh, wz, bz, g2, gd, att):
    e_blk = g.shape[0]
    cd = gc[:, :CW]
    d2 = jnp.sum(cd * cd, axis=1, keepdims=True)
    r = jnp.sqrt(d2)
    cutv = cut[0, 0]
    rbf = jnp.sqrt(2.0 / cutv) * jnp.sin(z0k[...] * (r / cutv)) / r
    ext = jnp.concatenate([r, rbf, jnp.zeros((e_blk, 7), _F32)], axis=1)
    h1 = g[...] + b1[...] + jnp.dot(ext, wext[...], preferred_element_type=_F32)
    m = jnp.dot(jax.nn.silu(h1), w2[...], preferred_element_type=_F32) + b2[...]
    hh = jax.nn.silu(jnp.dot(m, wh[...], preferred_element_type=_F32) + bh[...])
    z = jnp.dot(hh, wz[...], preferred_element_type=_F32) + bz[...]
    a = jax.nn.sigmoid(z[:, 0:1])
    g2[...] = m * a
    gd[...] = cd * (z[:, 1:2] / (r + 1.0))
    att[...] = a


def _edge(z0k, cut, g, gc, wext, b1, w2, b2, wh, bh, wz, bz, e_blk):
    e = g.shape[0]
    full = lambda shape: pl.BlockSpec(shape, lambda i: (0, 0))
    return pl.pallas_call(
        _edge_body,
        grid=(e // e_blk,),
        in_specs=[
            full((1, 8)),
            full((1, 1)),
            pl.BlockSpec((e_blk, C), lambda i: (i, 0)),
            pl.BlockSpec((e_blk, C), lambda i: (i, 0)),
            full((16, C)),
            full((1, C)),
            full((C, C)),
            full((1, C)),
            full((C, 2 * C)),
            full((1, 2 * C)),
            full((2 * C, 2)),
            full((1, 2)),
        ],
        out_specs=[
            pl.BlockSpec((e_blk, C), lambda i: (i, 0)),
            pl.BlockSpec((e_blk, CW), lambda i: (i, 0)),
            pl.BlockSpec((e_blk, 1), lambda i: (i, 0)),
        ],
        out_shape=[_sds((e, C)), _sds((e, CW)), _sds((e, 1))],
    )(z0k, cut, g, gc, wext, b1, w2, b2, wh, bh, wz, bz)


def _scatter_body(g2, gd, sidx, zf, zc, outf, outc, accf, accc, *scr):
    n_edges = sidx.shape[0]
    cid = lax.axis_index("c")
    sid = lax.axis_index("s")
    sets = (scr[:4], scr[4:])

    @pl.when(sid == 0)
    def _():
        pltpu.sync_copy(zf, accf)
        pltpu.sync_copy(zc, accc)

    plsc.subcore_barrier()
    wid = cid * NS + sid
    epw = n_edges // NW
    nchunk = epw // KCH

    def idx_fire(c, s):
        iv, buff, bufd, sem = s
        base = wid * epw + c * KCH
        pltpu.sync_copy(sidx.at[pl.ds(base, KCH)], iv)
        pltpu.async_copy(g2.at[pl.ds(base, KCH)], buff, sem)
        pltpu.async_copy(gd.at[pl.ds(base, KCH)], bufd, sem)

    def process(c, s):
        iv, buff, bufd, sem = s
        base = wid * epw + c * KCH
        pltpu.make_async_copy(g2.at[pl.ds(base, KCH)], buff, sem).wait()
        pltpu.make_async_copy(gd.at[pl.ds(base, KCH)], bufd, sem).wait()
        pltpu.sync_copy(buff, accf.at[iv], add=True)
        pltpu.sync_copy(bufd, accc.at[iv], add=True)

    idx_fire(0, sets[0])

    def chunk(i, carry):
        for par in range(2):
            @pl.when((i & 1) == par)
            def _():
                @pl.when(i + 1 < nchunk)
                def _():
                    idx_fire(i + 1, sets[1 - par])

                process(i, sets[par])

        return carry

    lax.fori_loop(0, nchunk, chunk, 0)
    plsc.subcore_barrier()

    @pl.when(sid == 0)
    def _():
        pltpu.sync_copy(accf, outf.at[cid])
        pltpu.sync_copy(accc, outc.at[cid])


def _scatter(g2, gd, sidx, zf, zc):
    n = zf.shape[0]
    one_set = [
        pltpu.VMEM((KCH,), jnp.int32),
        pltpu.VMEM((KCH, C), _F32),
        pltpu.VMEM((KCH, CW), _F32),
        pltpu.SemaphoreType.DMA,
    ]
    return pl.kernel(
        _scatter_body,
        out_type=(_sds((NC, n, C)), _sds((NC, n, CW))),
        mesh=_sc_mesh(),
        compiler_params=_SC_PARAMS,
        scratch_types=[
            pltpu.VMEM_SHARED((n, C), _F32),
            pltpu.VMEM_SHARED((n, CW), _F32),
        ] + one_set + one_set,
    )(g2, gd, sidx, zf, zc)


def _node_body(*refs):
    pfs = refs[:_NSLICE]
    pcs = refs[_NSLICE : 2 * _NSLICE]
    nf, co, wna, wnb, b1, w2, b2, onf, oc = refs[2 * _NSLICE :]
    m = pfs[0][0] + pfs[0][1]
    for p in pfs[1:]:
        m = m + p[0] + p[1]
    dc = pcs[0][0] + pcs[0][1]
    for p in pcs[1:]:
        dc = dc + p[0] + p[1]
    x = nf[...]
    h = jax.nn.silu(
        jnp.dot(x, wna[...], preferred_element_type=_F32)
        + jnp.dot(m, wnb[...], preferred_element_type=_F32)
        + b1[...]
    )
    onf[...] = jnp.dot(h, w2[...], preferred_element_type=_F32) + b2[...] + x
    oc[...] = co[...] + dc[:, :3]


def _node(pfs, pcs, nf, co, wna, wnb, b1, w2, b2, n_blk):
    n = nf.shape[0]
    full = lambda shape: pl.BlockSpec(shape, lambda i: (0, 0))
    return pl.pallas_call(
        _node_body,
        grid=(n // n_blk,),
        in_specs=(
            [pl.BlockSpec((NC, n_blk, C), lambda i: (0, i, 0))] * _NSLICE
            + [pl.BlockSpec((NC, n_blk, CW), lambda i: (0, i, 0))] * _NSLICE
            + [
                pl.BlockSpec((n_blk, C), lambda i: (i, 0)),
                pl.BlockSpec((n_blk, 3), lambda i: (i, 0)),
                full((C, C)),
                full((C, C)),
                full((1, C)),
                full((C, C)),
                full((1, C)),
            ]
        ),
        out_specs=[
            pl.BlockSpec((n_blk, C), lambda i: (i, 0)),
            pl.BlockSpec((n_blk, 3), lambda i: (i, 0)),
        ],
        out_shape=[_sds((n, C)), _sds((n, 3))],
    )(*pfs, *pcs, nf, co, wna, wnb, b1, w2, b2)


def kernel(node_feats, coordinates, edge_index, params):
    n = node_feats.shape[0]
    pe, pn, pa, px = params["phi_e"], params["phi_n"], params["att"], params["phi_x"]
    cut = params["bessel_cut_off"].reshape(1, 1).astype(_F32)
    z0k = params["z_0k"].reshape(1, -1).astype(_F32)
    nrbf = z0k.shape[1]

    w1 = pe["W1"]
    w1a, w1b = w1[:C], w1[C : 2 * C]
    wext = jnp.zeros((16, C), _F32).at[: nrbf + 1].set(w1[2 * C :])
    b1 = pe["b1"].reshape(1, C)
    w2, b2 = pe["W2"], pe["b2"].reshape(1, C)
    wh = jnp.concatenate([pa["W1"], px["W1"]], axis=1)
    bh = jnp.concatenate([pa["b1"], px["b1"]]).reshape(1, 2 * C)
    wz = (
        jnp.zeros((2 * C, 2), _F32)
        .at[:C, 0:1].set(pa["W2"])
        .at[C:, 1:2].set(px["W2"])
    )
    bz = jnp.concatenate([pa["b2"], px["b2"]]).reshape(1, 2)

    cp = jnp.pad(coordinates, ((0, 0), (0, CW - coordinates.shape[1])))
    sidx, ridx = edge_index[0], edge_index[1]
    e = sidx.shape[0]
    es = e // _NSLICE

    pa_t, pb_t = _proj(node_feats, w1a, w1b, n_blk=2000)
    zf = jnp.zeros((n, C), _F32)
    zc = jnp.zeros((n, CW), _F32)
    pfs, pcs, atts = [], [], []
    for k in range(_NSLICE):
        sl = slice(k * es, (k + 1) * es)
        g, gck = _gather(pa_t, pb_t, cp, sidx[sl], ridx[sl])
        g2, gd, att = _edge(
            z0k, cut, g, gck, wext, b1, w2, b2, wh, bh, wz, bz, e_blk=2560
        )
        pf, pc = _scatter(g2, gd, sidx[sl], zf, zc)
        pfs.append(pf)
        pcs.append(pc)
        atts.append(att)
    onf, oc = _node(
        pfs, pcs, node_feats, coordinates, pn["W1"][:C], pn["W1"][C:],
        pn["b1"].reshape(1, C), pn["W2"], pn["b2"].reshape(1, C), n_blk=1000,
    )
    return onf, oc, jnp.concatenate(atts, axis=0)
